# baseline (speedup 1.0000x reference)
import dataclasses
import functools

import jax
import jax.numpy as jnp
import numpy as np
from jax import lax
from jax.experimental import pallas as pl
from jax.experimental.pallas import tpu as pltpu
from jax.experimental.pallas import tpu_sc as plsc

NQ, NT, NC = 20000, 10000, 20000
NDIM = 128
H = 8
DK = 16
DFF = 4 * NDIM
HH = H // 2
HDIM = HH * DK
ACCW = 80
NSUB = 16
NSETS = 2
ZR = 25
BN = 1000


def _sc_cfg(n, E):
    return 16 if n == 20000 else 40


def _sc_body(n, E, *args):
    EB = _sc_cfg(n, E)
    (src_hbm, dst_hbm, klo, khi, qlo, qhi, vlo, vhi, out_lo, out_hi) = args[:10]
    rest = list(args[10:])
    idx = [(rest.pop(0), rest.pop(0)) for _ in range(NSETS)]
    gb = [(rest.pop(0), rest.pop(0), rest.pop(0)) for _ in range(NSETS)]
    rbuf = rest.pop(0)
    zbuf = rest.pop(0)
    acc = rest.pop(0)
    semi = [rest.pop(0) for _ in range(NSETS)]
    semg = [(rest.pop(0), rest.pop(0), rest.pop(0)) for _ in range(NSETS)]
    semz = rest.pop(0)

    c = lax.axis_index("c")
    s = lax.axis_index("s")
    ept = E // NSUB
    nblk = ept // EB
    rows_pt = n // NSUB
    zch = rows_pt // ZR
    assert nblk % 2 == 0
    ngroups = nblk // 2

    def half(ktab, qtab, vtab, out_hbm):
        tile_base = s * ept

        def fetch_idx(j, i):
            base = tile_base + j * EB
            pltpu.async_copy(src_hbm.at[pl.ds(base, EB)], idx[i][0], semi[i])
            pltpu.async_copy(dst_hbm.at[pl.ds(base, EB)], idx[i][1], semi[i])

        def wait_idx(j, i):
            base = tile_base + j * EB
            pltpu.make_async_copy(src_hbm.at[pl.ds(base, EB)], idx[i][0],
                                  semi[i]).wait()
            pltpu.make_async_copy(dst_hbm.at[pl.ds(base, EB)], idx[i][1],
                                  semi[i]).wait()

        def issue_gathers(i):
            srcv, dstv = idx[i]
            kb, qb, vb = gb[i]
            pltpu.async_copy(ktab.at[srcv], kb, semg[i][0])
            pltpu.async_copy(qtab.at[dstv], qb, semg[i][1])
            pltpu.async_copy(vtab.at[srcv], vb, semg[i][2])

        def wait_gathers(i):
            srcv, dstv = idx[i]
            kb, qb, vb = gb[i]
            pltpu.make_async_copy(ktab.at[srcv], kb, semg[i][0]).wait()
            pltpu.make_async_copy(qtab.at[dstv], qb, semg[i][1]).wait()
            pltpu.make_async_copy(vtab.at[srcv], vb, semg[i][2]).wait()

        def compute(i):
            kb, qb, vb = gb[i]
            rb = rbuf
            lanei = lax.iota(jnp.int32, DK)

            @pl.loop(0, EB)
            def _(e):
                zvec = jnp.zeros((DK,), jnp.float32)
                for h in range(HH):
                    kh = kb[e, pl.ds(DK * h, DK)]
                    qh = qb[e, pl.ds(DK * h, DK)]
                    sc = jnp.sum(kh * qh) * 0.25
                    sv = jnp.full((DK,), sc, jnp.float32)
                    sv = jnp.exp(jnp.clip(sv, -5.0, 5.0))
                    rb[e, pl.ds(DK * h, DK)] = sv * vb[e, pl.ds(DK * h, DK)]
                    zvec = zvec + jnp.where(lanei == h, sv, 0.0)
                rb[e, pl.ds(HDIM, DK)] = zvec

        def slot(j, i):
            @pl.when(j + 1 < nblk)
            def _():
                wait_idx(j + 1, 1 - i)
                issue_gathers(1 - i)

            wait_gathers(i)
            compute(i)
            pltpu.sync_copy(rbuf, acc.at[idx[i][1]], add=True)

            @pl.when(j + 2 < nblk)
            def _():
                fetch_idx(j + 2, i)

        @pl.loop(0, ZR)
        def _(r):
            for c5 in range(ACCW // 16):
                zbuf[r, pl.ds(16 * c5, 16)] = jnp.zeros((16,), jnp.float32)

        for zc0 in range(0, zch, 4):
            zcps = [pltpu.async_copy(zbuf,
                                     acc.at[pl.ds((s * zch + zc) * ZR, ZR)],
                                     semz)
                    for zc in range(zc0, min(zc0 + 4, zch))]
            for cp in zcps:
                cp.wait()

        fetch_idx(jnp.int32(0), 0)
        fetch_idx(jnp.int32(1), 1)
        plsc.subcore_barrier()
        wait_idx(jnp.int32(0), 0)
        issue_gathers(0)

        @pl.loop(0, ngroups)
        def _(t):
            slot(2 * t, 0)
            slot(2 * t + 1, 1)

        plsc.subcore_barrier()
        pltpu.sync_copy(acc.at[pl.ds(s * rows_pt, rows_pt)],
                        out_hbm.at[pl.ds(s * rows_pt, rows_pt)])

    @pl.when(c == 0)
    def _():
        half(klo, qlo, vlo, out_lo)

    @pl.when(c == 1)
    def _():
        half(khi, qhi, vhi, out_hi)


@functools.lru_cache(maxsize=None)
def _make_sc_kernel(n, E):
    EB = _sc_cfg(n, E)
    mesh = plsc.VectorSubcoreMesh(core_axis_name="c", subcore_axis_name="s",
                                  num_cores=2, num_subcores=NSUB)
    acc_t = jax.ShapeDtypeStruct((n, ACCW), jnp.float32)
    cp = pltpu.CompilerParams()
    if "needs_layout_passes" in pltpu.CompilerParams.__dataclass_fields__:
        cp = dataclasses.replace(cp, needs_layout_passes=False)
    if "use_tc_tiling_on_sc" in pltpu.CompilerParams.__dataclass_fields__:
        cp = dataclasses.replace(cp, use_tc_tiling_on_sc=False)
    return pl.kernel(
        functools.partial(_sc_body, n, E),
        out_type=(acc_t, acc_t),
        mesh=mesh,
        scratch_types=(
            [pltpu.VMEM((EB,), jnp.int32)] * (2 * NSETS)
            + [pltpu.VMEM((EB, HDIM), jnp.float32)] * (3 * NSETS)
            + [pltpu.VMEM((EB, ACCW), jnp.float32)]
            + [pltpu.VMEM((ZR, ACCW), jnp.float32),
               pltpu.VMEM_SHARED((n, ACCW), jnp.float32)]
            + [pltpu.SemaphoreType.DMA] * (NSETS + 3 * NSETS + 1)
        ),
        compiler_params=cp,
        interpret=False,
    )


def _proj_body(x_ref, wq_ref, bq_ref, wk_ref, wv_ref,
               qlo, qhi, klo, khi, vlo, vhi):
    xb = x_ref[...]
    q = jnp.dot(xb, wq_ref[...], preferred_element_type=jnp.float32) + bq_ref[...]
    k = jnp.dot(xb, wk_ref[...], preferred_element_type=jnp.float32)
    v = jnp.dot(xb, wv_ref[...], preferred_element_type=jnp.float32)
    qlo[...] = q[:, :HDIM]
    qhi[...] = q[:, HDIM:]
    klo[...] = k[:, :HDIM]
    khi[...] = k[:, HDIM:]
    vlo[...] = v[:, :HDIM]
    vhi[...] = v[:, HDIM:]


def _proj(xt, wq, bq, wk, wv):
    n = xt.shape[0]
    grid = (n // BN,)
    half_t = jax.ShapeDtypeStruct((n, HDIM), jnp.float32)
    row = pl.BlockSpec((BN, NDIM), lambda i: (i, 0))
    half = pl.BlockSpec((BN, HDIM), lambda i: (i, 0))
    w_sp = pl.BlockSpec((NDIM, NDIM), lambda i: (0, 0))
    b_sp = pl.BlockSpec((1, NDIM), lambda i: (0, 0))
    return pl.pallas_call(
        _proj_body,
        grid=grid,
        in_specs=[row, w_sp, b_sp, w_sp, w_sp],
        out_specs=[half] * 6,
        out_shape=[half_t] * 6,
    )(xt, wq, bq.reshape(1, NDIM), wk, wv)


def _ln(v, g, b):
    m = jnp.mean(v, axis=-1, keepdims=True)
    d = v - m
    var = jnp.mean(d * d, axis=-1, keepdims=True)
    return d * jax.lax.rsqrt(var + 1e-5) * g + b


def _post_body(x_ref, wv0_ref, z0_ref, wv1_ref, z1_ref,
               wo_ref, bo_ref, g1_ref, b1_ref, f1_ref, fb1_ref,
               f2_ref, fb2_ref, g2_ref, b2_ref, mw1_ref, mb1_ref, mw2_ref,
               o0_ref, o1_ref, ap_ref):
    xb = x_ref[...]
    ap = jnp.zeros((1, NDIM), jnp.float32)
    lane = lax.broadcasted_iota(jnp.int32, (1, NDIM), 1)
    for p, (wv_ref, z_ref, o_ref) in enumerate(
            [(wv0_ref, z0_ref, o0_ref), (wv1_ref, z1_ref, o1_ref)]):
        wv = wv_ref[...]
        z = z_ref[...]
        parts = [wv[:, DK * h:DK * (h + 1)] / (z[:, h:h + 1] + 1e-9)
                 for h in range(H)]
        oa = jnp.concatenate(parts, axis=1)
        o = _ln(xb + jnp.dot(oa, wo_ref[...], preferred_element_type=jnp.float32)
                + bo_ref[...], g1_ref[...], b1_ref[...])
        hmid = jnp.maximum(
            jnp.dot(o, f1_ref[...], preferred_element_type=jnp.float32)
            + fb1_ref[...], 0.0)
        o2 = _ln(o + jnp.dot(hmid, f2_ref[...], preferred_element_type=jnp.float32)
                 + fb2_ref[...], g2_ref[...], b2_ref[...])
        t = jnp.tanh(
            jnp.dot(o2, mw1_ref[...], preferred_element_type=jnp.float32)
            + mb1_ref[...])
        a_col = jnp.sum(t * mw2_ref[...], axis=-1, keepdims=True)
        o_ref[...] = o2
        ap = ap + jnp.where(lane == p, jnp.sum(a_col), 0.0)
    ap_ref[...] = ap.reshape(1, 1, NDIM)


def _post(xt, wv0, z0, wv1, z1, pp):
    n = xt.shape[0]
    nblk = n // BN
    row = pl.BlockSpec((BN, NDIM), lambda i: (i, 0))
    z_sp = pl.BlockSpec((BN, H), lambda i: (i, 0))
    const = lambda shape: pl.BlockSpec(shape, lambda i: (0, 0))
    out_row = jax.ShapeDtypeStruct((n, NDIM), jnp.float32)
    return pl.pallas_call(
        _post_body,
        grid=(nblk,),
        in_specs=[row, row, z_sp, row, z_sp,
                  const((NDIM, NDIM)), const((1, NDIM)), const((1, NDIM)),
                  const((1, NDIM)), const((NDIM, DFF)), const((1, DFF)),
                  const((DFF, NDIM)), const((1, NDIM)), const((1, NDIM)),
                  const((1, NDIM)), const((NDIM, NDIM)), const((1, NDIM)),
                  const((1, NDIM))],
        out_specs=[row, row, pl.BlockSpec((1, 1, NDIM), lambda i: (i, 0, 0))],
        out_shape=[out_row, out_row,
                   jax.ShapeDtypeStruct((nblk, 1, NDIM), jnp.float32)],
    )(xt, wv0, z0, wv1, z1,
      pp['Wo'], pp['bo'].reshape(1, NDIM), pp['ln1_g'].reshape(1, NDIM),
      pp['ln1_b'].reshape(1, NDIM), pp['fw1'], pp['fb1'].reshape(1, DFF),
      pp['fw2'], pp['fb2'].reshape(1, NDIM), pp['ln2_g'].reshape(1, NDIM),
      pp['ln2_b'].reshape(1, NDIM), pp['mp_w1'], pp['mp_b1'].reshape(1, NDIM),
      pp['mp_w2'].reshape(1, NDIM))


def _combine_body(o0_ref, o1_ref, w0_ref, w1_ref, out_ref):
    out_ref[...] = o0_ref[...] * w0_ref[...] + o1_ref[...] * w1_ref[...]


def _combine(o0, o1, w):
    n = o0.shape[0]
    row = pl.BlockSpec((BN, NDIM), lambda i: (i, 0))
    const = pl.BlockSpec((1, NDIM), lambda i: (0, 0))
    w0 = jnp.full((1, NDIM), w[0], jnp.float32)
    w1 = jnp.full((1, NDIM), w[1], jnp.float32)
    return pl.pallas_call(
        _combine_body,
        grid=(n // BN,),
        in_specs=[row, row, const, const],
        out_specs=row,
        out_shape=jax.ShapeDtypeStruct((n, NDIM), jnp.float32),
    )(o0, o1, w0, w1)


def _node_type(xt, pp, edge0, edge1, n):
    qlo, qhi, klo, khi, vlo, vhi = _proj(xt, pp['Wq'], pp['bq'],
                                         pp['Wk'], pp['Wv'])
    E = edge0.shape[1]
    sc = _make_sc_kernel(n, E)
    outs = []
    for edge in (edge0, edge1):
        acc_lo, acc_hi = sc(edge[0], edge[1], klo, khi, qlo, qhi, vlo, vhi)
        wv = jnp.concatenate([acc_lo[:, :HDIM], acc_hi[:, :HDIM]], axis=1)
        z = jnp.concatenate([acc_lo[:, HDIM:HDIM + HH],
                             acc_hi[:, HDIM:HDIM + HH]], axis=1)
        outs.append((wv, z))
    (wv0, z0), (wv1, z1) = outs
    o0, o1, apart = _post(xt, wv0, z0, wv1, z1, pp)
    a_tot = jnp.sum(apart, axis=(0, 1))
    w = jax.nn.softmax(jnp.stack([a_tot[0], a_tot[1]]) / n)
    return _combine(o0, o1, w)


def kernel(x, params, edge_q0, edge_q1, edge_t0, edge_t1, edge_c0, edge_c1):
    pq = {k: v[0] for k, v in params.items()}
    pt = {k: v[1] for k, v in params.items()}
    pc = {k: v[2] for k, v in params.items()}
    out_q = _node_type(x[:NQ], pq, edge_q0, edge_q1, NQ)
    out_t = _node_type(x[NQ:NQ + NT], pt, edge_t0, edge_t1, NT)
    out_c = _node_type(x[NQ + NT:], pc, edge_c0, edge_c1, NC)
    return jnp.concatenate([out_q, out_t, out_c], axis=0)


# speedup vs baseline: 3.1297x; 3.1297x over previous
import dataclasses
import functools

import jax
import jax.numpy as jnp
import numpy as np
from jax import lax
from jax.experimental import pallas as pl
from jax.experimental.pallas import tpu as pltpu
from jax.experimental.pallas import tpu_sc as plsc

NQ, NT, NC = 20000, 10000, 20000
NDIM = 128
H = 8
DK = 16
DFF = 4 * NDIM
HH = H // 2
HDIM = HH * DK
ACCW = 80
NSUB = 16
NSETS = 2
ZR = 25
BN = 1000


def _sc_cfg(n, E):
    return 16 if n == 20000 else 40


def _sc_body(n, E, *args):
    EB = _sc_cfg(n, E)
    (src_hbm, dst_hbm, klo, khi, qlo, qhi, vlo, vhi, out_lo, out_hi) = args[:10]
    rest = list(args[10:])
    idx = [(rest.pop(0), rest.pop(0)) for _ in range(NSETS)]
    gb = [(rest.pop(0), rest.pop(0), rest.pop(0)) for _ in range(NSETS)]
    rbuf = rest.pop(0)
    zbuf = rest.pop(0)
    acc = rest.pop(0)
    semi = [rest.pop(0) for _ in range(NSETS)]
    semg = [(rest.pop(0), rest.pop(0), rest.pop(0)) for _ in range(NSETS)]
    semz = rest.pop(0)

    c = lax.axis_index("c")
    s = lax.axis_index("s")
    ept = E // NSUB
    nblk = ept // EB
    rows_pt = n // NSUB
    zch = rows_pt // ZR
    assert nblk % 2 == 0
    ngroups = nblk // 2

    def half(ktab, qtab, vtab, out_hbm):
        tile_base = s * ept

        def fetch_idx(j, i):
            base = tile_base + j * EB
            pltpu.async_copy(src_hbm.at[pl.ds(base, EB)], idx[i][0], semi[i])
            pltpu.async_copy(dst_hbm.at[pl.ds(base, EB)], idx[i][1], semi[i])

        def wait_idx(j, i):
            base = tile_base + j * EB
            pltpu.make_async_copy(src_hbm.at[pl.ds(base, EB)], idx[i][0],
                                  semi[i]).wait()
            pltpu.make_async_copy(dst_hbm.at[pl.ds(base, EB)], idx[i][1],
                                  semi[i]).wait()

        def issue_gathers(i):
            srcv, dstv = idx[i]
            kb, qb, vb = gb[i]
            pltpu.async_copy(ktab.at[srcv], kb, semg[i][0])
            pltpu.async_copy(qtab.at[dstv], qb, semg[i][1])
            pltpu.async_copy(vtab.at[srcv], vb, semg[i][2])

        def wait_gathers(i):
            srcv, dstv = idx[i]
            kb, qb, vb = gb[i]
            pltpu.make_async_copy(ktab.at[srcv], kb, semg[i][0]).wait()
            pltpu.make_async_copy(qtab.at[dstv], qb, semg[i][1]).wait()
            pltpu.make_async_copy(vtab.at[srcv], vb, semg[i][2]).wait()

        def compute(i):
            kb, qb, vb = gb[i]
            rb = rbuf
            lanei = lax.iota(jnp.int32, DK)

            @plsc.parallel_loop(0, EB, step=1, unroll=4)
            def _(e):
                zvec = jnp.zeros((DK,), jnp.float32)
                for h in range(HH):
                    kh = kb[e, pl.ds(DK * h, DK)]
                    qh = qb[e, pl.ds(DK * h, DK)]
                    sc = jnp.sum(kh * qh) * 0.25
                    sv = jnp.full((DK,), sc, jnp.float32)
                    sv = jnp.exp(jnp.clip(sv, -5.0, 5.0))
                    rb[e, pl.ds(DK * h, DK)] = sv * vb[e, pl.ds(DK * h, DK)]
                    zvec = zvec + jnp.where(lanei == h, sv, 0.0)
                rb[e, pl.ds(HDIM, DK)] = zvec

        def slot(j, i):
            @pl.when(j + 1 < nblk)
            def _():
                wait_idx(j + 1, 1 - i)
                issue_gathers(1 - i)

            wait_gathers(i)
            compute(i)
            pltpu.sync_copy(rbuf, acc.at[idx[i][1]], add=True)

            @pl.when(j + 2 < nblk)
            def _():
                fetch_idx(j + 2, i)

        @pl.loop(0, ZR)
        def _(r):
            for c5 in range(ACCW // 16):
                zbuf[r, pl.ds(16 * c5, 16)] = jnp.zeros((16,), jnp.float32)

        for zc0 in range(0, zch, 4):
            zcps = [pltpu.async_copy(zbuf,
                                     acc.at[pl.ds((s * zch + zc) * ZR, ZR)],
                                     semz)
                    for zc in range(zc0, min(zc0 + 4, zch))]
            for cp in zcps:
                cp.wait()

        fetch_idx(jnp.int32(0), 0)
        fetch_idx(jnp.int32(1), 1)
        plsc.subcore_barrier()
        wait_idx(jnp.int32(0), 0)
        issue_gathers(0)

        @pl.loop(0, ngroups)
        def _(t):
            slot(2 * t, 0)
            slot(2 * t + 1, 1)

        plsc.subcore_barrier()
        pltpu.sync_copy(acc.at[pl.ds(s * rows_pt, rows_pt)],
                        out_hbm.at[pl.ds(s * rows_pt, rows_pt)])

    @pl.when(c == 0)
    def _():
        half(klo, qlo, vlo, out_lo)

    @pl.when(c == 1)
    def _():
        half(khi, qhi, vhi, out_hi)


@functools.lru_cache(maxsize=None)
def _make_sc_kernel(n, E):
    EB = _sc_cfg(n, E)
    mesh = plsc.VectorSubcoreMesh(core_axis_name="c", subcore_axis_name="s",
                                  num_cores=2, num_subcores=NSUB)
    acc_t = jax.ShapeDtypeStruct((n, ACCW), jnp.float32)
    cp = pltpu.CompilerParams()
    if "needs_layout_passes" in pltpu.CompilerParams.__dataclass_fields__:
        cp = dataclasses.replace(cp, needs_layout_passes=False)
    if "use_tc_tiling_on_sc" in pltpu.CompilerParams.__dataclass_fields__:
        cp = dataclasses.replace(cp, use_tc_tiling_on_sc=False)
    return pl.kernel(
        functools.partial(_sc_body, n, E),
        out_type=(acc_t, acc_t),
        mesh=mesh,
        scratch_types=(
            [pltpu.VMEM((EB,), jnp.int32)] * (2 * NSETS)
            + [pltpu.VMEM((EB, HDIM), jnp.float32)] * (3 * NSETS)
            + [pltpu.VMEM((EB, ACCW), jnp.float32)]
            + [pltpu.VMEM((ZR, ACCW), jnp.float32),
               pltpu.VMEM_SHARED((n, ACCW), jnp.float32)]
            + [pltpu.SemaphoreType.DMA] * (NSETS + 3 * NSETS + 1)
        ),
        compiler_params=cp,
        interpret=False,
    )


def _proj_body(x_ref, wq_ref, bq_ref, wk_ref, wv_ref,
               qlo, qhi, klo, khi, vlo, vhi):
    xb = x_ref[...]
    q = jnp.dot(xb, wq_ref[...], preferred_element_type=jnp.float32) + bq_ref[...]
    k = jnp.dot(xb, wk_ref[...], preferred_element_type=jnp.float32)
    v = jnp.dot(xb, wv_ref[...], preferred_element_type=jnp.float32)
    qlo[...] = q[:, :HDIM]
    qhi[...] = q[:, HDIM:]
    klo[...] = k[:, :HDIM]
    khi[...] = k[:, HDIM:]
    vlo[...] = v[:, :HDIM]
    vhi[...] = v[:, HDIM:]


def _proj(xt, wq, bq, wk, wv):
    n = xt.shape[0]
    grid = (n // BN,)
    half_t = jax.ShapeDtypeStruct((n, HDIM), jnp.float32)
    row = pl.BlockSpec((BN, NDIM), lambda i: (i, 0))
    half = pl.BlockSpec((BN, HDIM), lambda i: (i, 0))
    w_sp = pl.BlockSpec((NDIM, NDIM), lambda i: (0, 0))
    b_sp = pl.BlockSpec((1, NDIM), lambda i: (0, 0))
    return pl.pallas_call(
        _proj_body,
        grid=grid,
        in_specs=[row, w_sp, b_sp, w_sp, w_sp],
        out_specs=[half] * 6,
        out_shape=[half_t] * 6,
    )(xt, wq, bq.reshape(1, NDIM), wk, wv)


def _ln(v, g, b):
    m = jnp.mean(v, axis=-1, keepdims=True)
    d = v - m
    var = jnp.mean(d * d, axis=-1, keepdims=True)
    return d * jax.lax.rsqrt(var + 1e-5) * g + b


def _post_body(x_ref, wv0_ref, z0_ref, wv1_ref, z1_ref,
               wo_ref, bo_ref, g1_ref, b1_ref, f1_ref, fb1_ref,
               f2_ref, fb2_ref, g2_ref, b2_ref, mw1_ref, mb1_ref, mw2_ref,
               o0_ref, o1_ref, ap_ref):
    xb = x_ref[...]
    ap = jnp.zeros((1, NDIM), jnp.float32)
    lane = lax.broadcasted_iota(jnp.int32, (1, NDIM), 1)
    for p, (wv_ref, z_ref, o_ref) in enumerate(
            [(wv0_ref, z0_ref, o0_ref), (wv1_ref, z1_ref, o1_ref)]):
        wv = wv_ref[...]
        z = z_ref[...]
        parts = [wv[:, DK * h:DK * (h + 1)] / (z[:, h:h + 1] + 1e-9)
                 for h in range(H)]
        oa = jnp.concatenate(parts, axis=1)
        o = _ln(xb + jnp.dot(oa, wo_ref[...], preferred_element_type=jnp.float32)
                + bo_ref[...], g1_ref[...], b1_ref[...])
        hmid = jnp.maximum(
            jnp.dot(o, f1_ref[...], preferred_element_type=jnp.float32)
            + fb1_ref[...], 0.0)
        o2 = _ln(o + jnp.dot(hmid, f2_ref[...], preferred_element_type=jnp.float32)
                 + fb2_ref[...], g2_ref[...], b2_ref[...])
        t = jnp.tanh(
            jnp.dot(o2, mw1_ref[...], preferred_element_type=jnp.float32)
            + mb1_ref[...])
        a_col = jnp.sum(t * mw2_ref[...], axis=-1, keepdims=True)
        o_ref[...] = o2
        ap = ap + jnp.where(lane == p, jnp.sum(a_col), 0.0)
    ap_ref[...] = ap.reshape(1, 1, NDIM)


def _post(xt, wv0, z0, wv1, z1, pp):
    n = xt.shape[0]
    nblk = n // BN
    row = pl.BlockSpec((BN, NDIM), lambda i: (i, 0))
    z_sp = pl.BlockSpec((BN, H), lambda i: (i, 0))
    const = lambda shape: pl.BlockSpec(shape, lambda i: (0, 0))
    out_row = jax.ShapeDtypeStruct((n, NDIM), jnp.float32)
    return pl.pallas_call(
        _post_body,
        grid=(nblk,),
        in_specs=[row, row, z_sp, row, z_sp,
                  const((NDIM, NDIM)), const((1, NDIM)), const((1, NDIM)),
                  const((1, NDIM)), const((NDIM, DFF)), const((1, DFF)),
                  const((DFF, NDIM)), const((1, NDIM)), const((1, NDIM)),
                  const((1, NDIM)), const((NDIM, NDIM)), const((1, NDIM)),
                  const((1, NDIM))],
        out_specs=[row, row, pl.BlockSpec((1, 1, NDIM), lambda i: (i, 0, 0))],
        out_shape=[out_row, out_row,
                   jax.ShapeDtypeStruct((nblk, 1, NDIM), jnp.float32)],
    )(xt, wv0, z0, wv1, z1,
      pp['Wo'], pp['bo'].reshape(1, NDIM), pp['ln1_g'].reshape(1, NDIM),
      pp['ln1_b'].reshape(1, NDIM), pp['fw1'], pp['fb1'].reshape(1, DFF),
      pp['fw2'], pp['fb2'].reshape(1, NDIM), pp['ln2_g'].reshape(1, NDIM),
      pp['ln2_b'].reshape(1, NDIM), pp['mp_w1'], pp['mp_b1'].reshape(1, NDIM),
      pp['mp_w2'].reshape(1, NDIM))


def _combine_body(o0_ref, o1_ref, w0_ref, w1_ref, out_ref):
    out_ref[...] = o0_ref[...] * w0_ref[...] + o1_ref[...] * w1_ref[...]


def _combine(o0, o1, w):
    n = o0.shape[0]
    row = pl.BlockSpec((BN, NDIM), lambda i: (i, 0))
    const = pl.BlockSpec((1, NDIM), lambda i: (0, 0))
    w0 = jnp.full((1, NDIM), w[0], jnp.float32)
    w1 = jnp.full((1, NDIM), w[1], jnp.float32)
    return pl.pallas_call(
        _combine_body,
        grid=(n // BN,),
        in_specs=[row, row, const, const],
        out_specs=row,
        out_shape=jax.ShapeDtypeStruct((n, NDIM), jnp.float32),
    )(o0, o1, w0, w1)


def _node_type(xt, pp, edge0, edge1, n):
    qlo, qhi, klo, khi, vlo, vhi = _proj(xt, pp['Wq'], pp['bq'],
                                         pp['Wk'], pp['Wv'])
    E = edge0.shape[1]
    sc = _make_sc_kernel(n, E)
    outs = []
    for edge in (edge0, edge1):
        acc_lo, acc_hi = sc(edge[0], edge[1], klo, khi, qlo, qhi, vlo, vhi)
        wv = jnp.concatenate([acc_lo[:, :HDIM], acc_hi[:, :HDIM]], axis=1)
        z = jnp.concatenate([acc_lo[:, HDIM:HDIM + HH],
                             acc_hi[:, HDIM:HDIM + HH]], axis=1)
        outs.append((wv, z))
    (wv0, z0), (wv1, z1) = outs
    o0, o1, apart = _post(xt, wv0, z0, wv1, z1, pp)
    a_tot = jnp.sum(apart, axis=(0, 1))
    w = jax.nn.softmax(jnp.stack([a_tot[0], a_tot[1]]) / n)
    return _combine(o0, o1, w)


def kernel(x, params, edge_q0, edge_q1, edge_t0, edge_t1, edge_c0, edge_c1):
    pq = {k: v[0] for k, v in params.items()}
    pt = {k: v[1] for k, v in params.items()}
    pc = {k: v[2] for k, v in params.items()}
    out_q = _node_type(x[:NQ], pq, edge_q0, edge_q1, NQ)
    out_t = _node_type(x[NQ:NQ + NT], pt, edge_t0, edge_t1, NT)
    out_c = _node_type(x[NQ + NT:], pc, edge_c0, edge_c1, NC)
    return jnp.concatenate([out_q, out_t, out_c], axis=0)


# speedup vs baseline: 4.8966x; 1.5646x over previous
import dataclasses
import functools

import jax
import jax.numpy as jnp
import numpy as np
from jax import lax
from jax.experimental import pallas as pl
from jax.experimental.pallas import tpu as pltpu
from jax.experimental.pallas import tpu_sc as plsc

NQ, NT, NC = 20000, 10000, 20000
NDIM = 128
H = 8
DK = 16
DFF = 4 * NDIM
HH = H // 2
HDIM = HH * DK
ACCW = 80
NSUB = 16
NSETS = 2
ZR = 25
BN = 1000


def _sc_cfg(n, E):
    return 40


def _sc_body(n, E, *args):
    EB = _sc_cfg(n, E)
    (src_hbm, dst_hbm, klo, khi, qlo, qhi, vlo, vhi, out_lo, out_hi) = args[:10]
    rest = list(args[10:])
    idx = [(rest.pop(0), rest.pop(0)) for _ in range(NSETS)]
    gb = [(rest.pop(0), rest.pop(0), rest.pop(0)) for _ in range(NSETS)]
    rbuf = rest.pop(0)
    zbuf = rest.pop(0)
    acc = rest.pop(0)
    semi = [rest.pop(0) for _ in range(NSETS)]
    semg = [(rest.pop(0), rest.pop(0), rest.pop(0)) for _ in range(NSETS)]
    semz = rest.pop(0)

    c = lax.axis_index("c")
    s = lax.axis_index("s")
    ept = E // NSUB
    nblk = ept // EB
    rows_pt = n // NSUB
    zch = rows_pt // ZR
    assert nblk % 2 == 0
    ngroups = nblk // 2

    def half(ktab, qtab, vtab, out_hbm):
        tile_base = s * ept

        def fetch_idx(j, i):
            base = tile_base + j * EB
            pltpu.async_copy(src_hbm.at[pl.ds(base, EB)], idx[i][0], semi[i])
            pltpu.async_copy(dst_hbm.at[pl.ds(base, EB)], idx[i][1], semi[i])

        def wait_idx(j, i):
            base = tile_base + j * EB
            pltpu.make_async_copy(src_hbm.at[pl.ds(base, EB)], idx[i][0],
                                  semi[i]).wait()
            pltpu.make_async_copy(dst_hbm.at[pl.ds(base, EB)], idx[i][1],
                                  semi[i]).wait()

        def issue_gathers(i):
            srcv, dstv = idx[i]
            kb, qb, vb = gb[i]
            pltpu.async_copy(ktab.at[srcv], kb, semg[i][0])
            pltpu.async_copy(qtab.at[dstv], qb, semg[i][1])
            pltpu.async_copy(vtab.at[srcv], vb, semg[i][2])

        def wait_gathers(i):
            srcv, dstv = idx[i]
            kb, qb, vb = gb[i]
            pltpu.make_async_copy(ktab.at[srcv], kb, semg[i][0]).wait()
            pltpu.make_async_copy(qtab.at[dstv], qb, semg[i][1]).wait()
            pltpu.make_async_copy(vtab.at[srcv], vb, semg[i][2]).wait()

        def compute(i):
            kb, qb, vb = gb[i]
            rb = rbuf
            lanei = lax.iota(jnp.int32, DK)

            @plsc.parallel_loop(0, EB, step=1, unroll=8)
            def _(e):
                zvec = jnp.zeros((DK,), jnp.float32)
                for h in range(HH):
                    kh = kb[e, pl.ds(DK * h, DK)]
                    qh = qb[e, pl.ds(DK * h, DK)]
                    sc = jnp.sum(kh * qh) * 0.25
                    sv = jnp.full((DK,), sc, jnp.float32)
                    sv = jnp.exp(jnp.clip(sv, -5.0, 5.0))
                    rb[e, pl.ds(DK * h, DK)] = sv * vb[e, pl.ds(DK * h, DK)]
                    zvec = zvec + jnp.where(lanei == h, sv, 0.0)
                rb[e, pl.ds(HDIM, DK)] = zvec

        def slot(j, i):
            @pl.when(j + 1 < nblk)
            def _():
                wait_idx(j + 1, 1 - i)
                issue_gathers(1 - i)

            wait_gathers(i)
            compute(i)
            pltpu.sync_copy(rbuf, acc.at[idx[i][1]], add=True)

            @pl.when(j + 2 < nblk)
            def _():
                fetch_idx(j + 2, i)

        @pl.loop(0, ZR)
        def _(r):
            for c5 in range(ACCW // 16):
                zbuf[r, pl.ds(16 * c5, 16)] = jnp.zeros((16,), jnp.float32)

        for zc0 in range(0, zch, 4):
            zcps = [pltpu.async_copy(zbuf,
                                     acc.at[pl.ds((s * zch + zc) * ZR, ZR)],
                                     semz)
                    for zc in range(zc0, min(zc0 + 4, zch))]
            for cp in zcps:
                cp.wait()

        fetch_idx(jnp.int32(0), 0)
        fetch_idx(jnp.int32(1), 1)
        plsc.subcore_barrier()
        wait_idx(jnp.int32(0), 0)
        issue_gathers(0)

        @pl.loop(0, ngroups)
        def _(t):
            slot(2 * t, 0)
            slot(2 * t + 1, 1)

        plsc.subcore_barrier()
        pltpu.sync_copy(acc.at[pl.ds(s * rows_pt, rows_pt)],
                        out_hbm.at[pl.ds(s * rows_pt, rows_pt)])

    @pl.when(c == 0)
    def _():
        half(klo, qlo, vlo, out_lo)

    @pl.when(c == 1)
    def _():
        half(khi, qhi, vhi, out_hi)


@functools.lru_cache(maxsize=None)
def _make_sc_kernel(n, E):
    EB = _sc_cfg(n, E)
    mesh = plsc.VectorSubcoreMesh(core_axis_name="c", subcore_axis_name="s",
                                  num_cores=2, num_subcores=NSUB)
    acc_t = jax.ShapeDtypeStruct((n, ACCW), jnp.float32)
    cp = pltpu.CompilerParams()
    if "needs_layout_passes" in pltpu.CompilerParams.__dataclass_fields__:
        cp = dataclasses.replace(cp, needs_layout_passes=False)
    if "use_tc_tiling_on_sc" in pltpu.CompilerParams.__dataclass_fields__:
        cp = dataclasses.replace(cp, use_tc_tiling_on_sc=False)
    return pl.kernel(
        functools.partial(_sc_body, n, E),
        out_type=(acc_t, acc_t),
        mesh=mesh,
        scratch_types=(
            [pltpu.VMEM((EB,), jnp.int32)] * (2 * NSETS)
            + [pltpu.VMEM((EB, HDIM), jnp.float32)] * (3 * NSETS)
            + [pltpu.VMEM((EB, ACCW), jnp.float32)]
            + [pltpu.VMEM((ZR, ACCW), jnp.float32),
               pltpu.VMEM_SHARED((n, ACCW), jnp.float32)]
            + [pltpu.SemaphoreType.DMA] * (NSETS + 3 * NSETS + 1)
        ),
        compiler_params=cp,
        interpret=False,
    )


def _proj_body(x_ref, wq_ref, bq_ref, wk_ref, wv_ref,
               qlo, qhi, klo, khi, vlo, vhi):
    xb = x_ref[...]
    q = jnp.dot(xb, wq_ref[...], preferred_element_type=jnp.float32) + bq_ref[...]
    k = jnp.dot(xb, wk_ref[...], preferred_element_type=jnp.float32)
    v = jnp.dot(xb, wv_ref[...], preferred_element_type=jnp.float32)
    qlo[...] = q[:, :HDIM]
    qhi[...] = q[:, HDIM:]
    klo[...] = k[:, :HDIM]
    khi[...] = k[:, HDIM:]
    vlo[...] = v[:, :HDIM]
    vhi[...] = v[:, HDIM:]


def _proj(xt, wq, bq, wk, wv):
    n = xt.shape[0]
    grid = (n // BN,)
    half_t = jax.ShapeDtypeStruct((n, HDIM), jnp.float32)
    row = pl.BlockSpec((BN, NDIM), lambda i: (i, 0))
    half = pl.BlockSpec((BN, HDIM), lambda i: (i, 0))
    w_sp = pl.BlockSpec((NDIM, NDIM), lambda i: (0, 0))
    b_sp = pl.BlockSpec((1, NDIM), lambda i: (0, 0))
    return pl.pallas_call(
        _proj_body,
        grid=grid,
        in_specs=[row, w_sp, b_sp, w_sp, w_sp],
        out_specs=[half] * 6,
        out_shape=[half_t] * 6,
    )(xt, wq, bq.reshape(1, NDIM), wk, wv)


def _ln(v, g, b):
    m = jnp.mean(v, axis=-1, keepdims=True)
    d = v - m
    var = jnp.mean(d * d, axis=-1, keepdims=True)
    return d * jax.lax.rsqrt(var + 1e-5) * g + b


def _post_body(x_ref, wv0_ref, z0_ref, wv1_ref, z1_ref,
               wo_ref, bo_ref, g1_ref, b1_ref, f1_ref, fb1_ref,
               f2_ref, fb2_ref, g2_ref, b2_ref, mw1_ref, mb1_ref, mw2_ref,
               o0_ref, o1_ref, ap_ref):
    xb = x_ref[...]
    ap = jnp.zeros((1, NDIM), jnp.float32)
    lane = lax.broadcasted_iota(jnp.int32, (1, NDIM), 1)
    for p, (wv_ref, z_ref, o_ref) in enumerate(
            [(wv0_ref, z0_ref, o0_ref), (wv1_ref, z1_ref, o1_ref)]):
        wv = wv_ref[...]
        z = z_ref[...]
        parts = [wv[:, DK * h:DK * (h + 1)] / (z[:, h:h + 1] + 1e-9)
                 for h in range(H)]
        oa = jnp.concatenate(parts, axis=1)
        o = _ln(xb + jnp.dot(oa, wo_ref[...], preferred_element_type=jnp.float32)
                + bo_ref[...], g1_ref[...], b1_ref[...])
        hmid = jnp.maximum(
            jnp.dot(o, f1_ref[...], preferred_element_type=jnp.float32)
            + fb1_ref[...], 0.0)
        o2 = _ln(o + jnp.dot(hmid, f2_ref[...], preferred_element_type=jnp.float32)
                 + fb2_ref[...], g2_ref[...], b2_ref[...])
        t = jnp.tanh(
            jnp.dot(o2, mw1_ref[...], preferred_element_type=jnp.float32)
            + mb1_ref[...])
        a_col = jnp.sum(t * mw2_ref[...], axis=-1, keepdims=True)
        o_ref[...] = o2
        ap = ap + jnp.where(lane == p, jnp.sum(a_col), 0.0)
    ap_ref[...] = ap.reshape(1, 1, NDIM)


def _post(xt, wv0, z0, wv1, z1, pp):
    n = xt.shape[0]
    nblk = n // BN
    row = pl.BlockSpec((BN, NDIM), lambda i: (i, 0))
    z_sp = pl.BlockSpec((BN, H), lambda i: (i, 0))
    const = lambda shape: pl.BlockSpec(shape, lambda i: (0, 0))
    out_row = jax.ShapeDtypeStruct((n, NDIM), jnp.float32)
    return pl.pallas_call(
        _post_body,
        grid=(nblk,),
        in_specs=[row, row, z_sp, row, z_sp,
                  const((NDIM, NDIM)), const((1, NDIM)), const((1, NDIM)),
                  const((1, NDIM)), const((NDIM, DFF)), const((1, DFF)),
                  const((DFF, NDIM)), const((1, NDIM)), const((1, NDIM)),
                  const((1, NDIM)), const((NDIM, NDIM)), const((1, NDIM)),
                  const((1, NDIM))],
        out_specs=[row, row, pl.BlockSpec((1, 1, NDIM), lambda i: (i, 0, 0))],
        out_shape=[out_row, out_row,
                   jax.ShapeDtypeStruct((nblk, 1, NDIM), jnp.float32)],
    )(xt, wv0, z0, wv1, z1,
      pp['Wo'], pp['bo'].reshape(1, NDIM), pp['ln1_g'].reshape(1, NDIM),
      pp['ln1_b'].reshape(1, NDIM), pp['fw1'], pp['fb1'].reshape(1, DFF),
      pp['fw2'], pp['fb2'].reshape(1, NDIM), pp['ln2_g'].reshape(1, NDIM),
      pp['ln2_b'].reshape(1, NDIM), pp['mp_w1'], pp['mp_b1'].reshape(1, NDIM),
      pp['mp_w2'].reshape(1, NDIM))


def _combine_body(o0_ref, o1_ref, w0_ref, w1_ref, out_ref):
    out_ref[...] = o0_ref[...] * w0_ref[...] + o1_ref[...] * w1_ref[...]


def _combine(o0, o1, w):
    n = o0.shape[0]
    row = pl.BlockSpec((BN, NDIM), lambda i: (i, 0))
    const = pl.BlockSpec((1, NDIM), lambda i: (0, 0))
    w0 = jnp.full((1, NDIM), w[0], jnp.float32)
    w1 = jnp.full((1, NDIM), w[1], jnp.float32)
    return pl.pallas_call(
        _combine_body,
        grid=(n // BN,),
        in_specs=[row, row, const, const],
        out_specs=row,
        out_shape=jax.ShapeDtypeStruct((n, NDIM), jnp.float32),
    )(o0, o1, w0, w1)


def _node_type(xt, pp, edge0, edge1, n):
    qlo, qhi, klo, khi, vlo, vhi = _proj(xt, pp['Wq'], pp['bq'],
                                         pp['Wk'], pp['Wv'])
    E = edge0.shape[1]
    sc = _make_sc_kernel(n, E)
    outs = []
    for edge in (edge0, edge1):
        acc_lo, acc_hi = sc(edge[0], edge[1], klo, khi, qlo, qhi, vlo, vhi)
        wv = jnp.concatenate([acc_lo[:, :HDIM], acc_hi[:, :HDIM]], axis=1)
        z = jnp.concatenate([acc_lo[:, HDIM:HDIM + HH],
                             acc_hi[:, HDIM:HDIM + HH]], axis=1)
        outs.append((wv, z))
    (wv0, z0), (wv1, z1) = outs
    o0, o1, apart = _post(xt, wv0, z0, wv1, z1, pp)
    a_tot = jnp.sum(apart, axis=(0, 1))
    w = jax.nn.softmax(jnp.stack([a_tot[0], a_tot[1]]) / n)
    return _combine(o0, o1, w)


def kernel(x, params, edge_q0, edge_q1, edge_t0, edge_t1, edge_c0, edge_c1):
    pq = {k: v[0] for k, v in params.items()}
    pt = {k: v[1] for k, v in params.items()}
    pc = {k: v[2] for k, v in params.items()}
    out_q = _node_type(x[:NQ], pq, edge_q0, edge_q1, NQ)
    out_t = _node_type(x[NQ:NQ + NT], pt, edge_t0, edge_t1, NT)
    out_c = _node_type(x[NQ + NT:], pc, edge_c0, edge_c1, NC)
    return jnp.concatenate([out_q, out_t, out_c], axis=0)
